# baseline (speedup 1.0000x reference)
import functools

import jax
import jax.numpy as jnp
from jax import lax
from jax.experimental import pallas as pl
from jax.experimental.pallas import tpu as pltpu
from jax.experimental.pallas import tpu_sc as plsc

_D = 768
_NE = 64
_BT = 4096
_T = 32768

_NW = 32
_ROWS_PER_W = _T // _NW
_CHUNK = 512
_GROUP = 16


def _matmul_body(x_ref, w_ref, b_ref, o_ref):
    s = lax.dot_general(
        x_ref[...], w_ref[...],
        (((1,), (1,)), ((), ())),
        preferred_element_type=jnp.float32,
    )
    o_ref[...] = s + b_ref[...]


def _logits(x, W, b):
    t = x.shape[0]
    b2 = b.reshape(1, _NE)
    return pl.pallas_call(
        _matmul_body,
        grid=(t // _BT,),
        in_specs=[
            pl.BlockSpec((_BT, _D), lambda i: (i, 0)),
            pl.BlockSpec((_NE, _D), lambda i: (0, 0)),
            pl.BlockSpec((1, _NE), lambda i: (0, 0)),
        ],
        out_specs=pl.BlockSpec((_BT, _NE), lambda i: (i, 0)),
        out_shape=jax.ShapeDtypeStruct((t, _NE), jnp.float32),
    )(x, W, b2)


def _route_body(s_hbm, out_hbm, buf, obuf):
    wid = lax.axis_index("s") * 2 + lax.axis_index("c")
    lane = jnp.arange(_GROUP, dtype=jnp.int32)
    zeros = jnp.zeros((_GROUP,), jnp.float32)
    neginf = jnp.full((_GROUP,), -jnp.inf, jnp.float32)

    def chunk_step(c, carry):
        base = (wid * _ROWS_PER_W + c * _CHUNK) * _NE
        pltpu.sync_copy(s_hbm.at[pl.ds(base, _CHUNK * _NE)], buf)

        def group_step(g, inner):
            gbase = g * _GROUP * _NE
            flat_rows = gbase + lane * _NE
            for q in range(_GROUP * _NE // _GROUP):
                obuf[pl.ds(gbase + q * _GROUP, _GROUP)] = zeros
            m1 = neginf
            m2 = neginf
            i1 = jnp.zeros((_GROUP,), jnp.int32)
            i2 = jnp.zeros((_GROUP,), jnp.int32)
            for j in range(_NE):
                v = plsc.load_gather(buf, [flat_rows + j])
                jv = jnp.full((_GROUP,), j, jnp.int32)
                b1 = v > m1
                b2 = v > m2
                m2 = jnp.where(b1, m1, jnp.where(b2, v, m2))
                i2 = jnp.where(b1, i1, jnp.where(b2, jv, i2))
                m1 = jnp.where(b1, v, m1)
                i1 = jnp.where(b1, jv, i1)
            r = jnp.exp(m2 - m1)
            recip = 1.0 / (1.0 + r)
            plsc.store_scatter(obuf, [flat_rows + i1], recip)
            plsc.store_scatter(obuf, [flat_rows + i2], r * recip)
            return inner

        lax.fori_loop(0, _CHUNK // _GROUP, group_step, 0)
        pltpu.sync_copy(obuf, out_hbm.at[pl.ds(base, _CHUNK * _NE)])
        return carry

    lax.fori_loop(0, _ROWS_PER_W // _CHUNK, chunk_step, 0)


def _route(s):
    mesh = plsc.VectorSubcoreMesh(core_axis_name="c", subcore_axis_name="s")
    f = functools.partial(
        pl.kernel,
        mesh=mesh,
        out_type=jax.ShapeDtypeStruct((_T * _NE,), jnp.float32),
        scratch_types=[
            pltpu.VMEM((_CHUNK * _NE,), jnp.float32),
            pltpu.VMEM((_CHUNK * _NE,), jnp.float32),
        ],
        compiler_params=pltpu.CompilerParams(needs_layout_passes=False),
    )(_route_body)
    return f(s.reshape(_T * _NE)).reshape(_T, _NE)


def kernel(x, W, b):
    return _route(_logits(x, W, b))


# speedup vs baseline: 1.1309x; 1.1309x over previous
import functools

import jax
import jax.numpy as jnp
from jax import lax
from jax.experimental import pallas as pl
from jax.experimental.pallas import tpu as pltpu
from jax.experimental.pallas import tpu_sc as plsc

_D = 768
_NE = 64
_BT = 4096
_T = 32768

_NW = 32
_ROWS_PER_W = _T // _NW
_CHUNK = 512
_GROUP = 16


def _matmul_body(x_ref, w_ref, b_ref, o_ref):
    s = lax.dot_general(
        x_ref[...], w_ref[...],
        (((1,), (1,)), ((), ())),
        preferred_element_type=jnp.float32,
    )
    o_ref[...] = s + b_ref[...]


def _logits(x, W, b):
    t = x.shape[0]
    b2 = b.reshape(1, _NE)
    return pl.pallas_call(
        _matmul_body,
        grid=(t // _BT,),
        in_specs=[
            pl.BlockSpec((_BT, _D), lambda i: (i, 0)),
            pl.BlockSpec((_NE, _D), lambda i: (0, 0)),
            pl.BlockSpec((1, _NE), lambda i: (0, 0)),
        ],
        out_specs=pl.BlockSpec((_BT, _NE), lambda i: (i, 0)),
        out_shape=jax.ShapeDtypeStruct((t, _NE), jnp.float32),
    )(x, W, b2)


def _route_body(s_hbm, out_hbm, buf, obuf):
    wid = lax.axis_index("s") * 2 + lax.axis_index("c")
    lane = jnp.arange(_GROUP, dtype=jnp.int32)
    zeros = jnp.zeros((_GROUP,), jnp.float32)
    neginf = jnp.full((_GROUP,), -jnp.inf, jnp.float32)

    def chunk_step(c, carry):
        base = (wid * _ROWS_PER_W + c * _CHUNK) * _NE
        pltpu.sync_copy(s_hbm.at[pl.ds(base, _CHUNK * _NE)], buf)

        def group_step(g, inner):
            gbase = g * _GROUP * _NE
            flat_rows = gbase + lane * _NE
            for q in range(_GROUP * _NE // _GROUP):
                obuf[pl.ds(gbase + q * _GROUP, _GROUP)] = zeros
            m1 = neginf
            m2 = neginf
            i1 = jnp.zeros((_GROUP,), jnp.int32)
            i2 = jnp.zeros((_GROUP,), jnp.int32)
            for j in range(_NE):
                jv = (lane + j) & (_NE - 1)
                v = plsc.load_gather(buf, [flat_rows + jv])
                b1 = (v > m1) | ((v == m1) & (jv < i1))
                b2 = (v > m2) | ((v == m2) & (jv < i2))
                m2 = jnp.where(b1, m1, jnp.where(b2, v, m2))
                i2 = jnp.where(b1, i1, jnp.where(b2, jv, i2))
                m1 = jnp.where(b1, v, m1)
                i1 = jnp.where(b1, jv, i1)
            r = jnp.exp(m2 - m1)
            recip = 1.0 / (1.0 + r)
            plsc.store_scatter(obuf, [flat_rows + i1], recip)
            plsc.store_scatter(obuf, [flat_rows + i2], r * recip)
            return inner

        lax.fori_loop(0, _CHUNK // _GROUP, group_step, 0)
        pltpu.sync_copy(obuf, out_hbm.at[pl.ds(base, _CHUNK * _NE)])
        return carry

    lax.fori_loop(0, _ROWS_PER_W // _CHUNK, chunk_step, 0)


def _route(s):
    mesh = plsc.VectorSubcoreMesh(core_axis_name="c", subcore_axis_name="s")
    f = functools.partial(
        pl.kernel,
        mesh=mesh,
        out_type=jax.ShapeDtypeStruct((_T * _NE,), jnp.float32),
        scratch_types=[
            pltpu.VMEM((_CHUNK * _NE,), jnp.float32),
            pltpu.VMEM((_CHUNK * _NE,), jnp.float32),
        ],
        compiler_params=pltpu.CompilerParams(needs_layout_passes=False),
    )(_route_body)
    return f(s.reshape(_T * _NE)).reshape(_T, _NE)


def kernel(x, W, b):
    return _route(_logits(x, W, b))


# speedup vs baseline: 1.3564x; 1.1993x over previous
import functools

import jax
import jax.numpy as jnp
from jax import lax
from jax.experimental import pallas as pl
from jax.experimental.pallas import tpu as pltpu
from jax.experimental.pallas import tpu_sc as plsc

_D = 768
_NE = 64
_BT = 4096
_T = 32768

_NW = 32
_ROWS_PER_W = _T // _NW
_CHUNK = 256
_GROUP = 16


def _matmul_body(x_ref, w_ref, b_ref, o_ref):
    s = lax.dot_general(
        x_ref[...], w_ref[...],
        (((1,), (1,)), ((), ())),
        preferred_element_type=jnp.float32,
    )
    o_ref[...] = s + b_ref[...]


def _logits(x, W, b):
    t = x.shape[0]
    b2 = b.reshape(1, _NE)
    return pl.pallas_call(
        _matmul_body,
        grid=(t // _BT,),
        in_specs=[
            pl.BlockSpec((_BT, _D), lambda i: (i, 0)),
            pl.BlockSpec((_NE, _D), lambda i: (0, 0)),
            pl.BlockSpec((1, _NE), lambda i: (0, 0)),
        ],
        out_specs=pl.BlockSpec((_BT, _NE), lambda i: (i, 0)),
        out_shape=jax.ShapeDtypeStruct((t, _NE), jnp.float32),
    )(x, W, b2)


def _route_body(s_hbm, out_hbm, buf, obuf):
    wid = lax.axis_index("s") * 2 + lax.axis_index("c")
    lane = jnp.arange(_GROUP, dtype=jnp.int32)
    zeros = jnp.zeros((_GROUP,), jnp.float32)
    neginf = jnp.full((_GROUP,), -jnp.inf, jnp.float32)

    def chunk_step(c, carry):
        base = wid * _ROWS_PER_W + c * _CHUNK
        pltpu.sync_copy(s_hbm.at[pl.ds(base, _CHUNK)], buf)

        def group_step(g, inner):
            rows = g * _GROUP + lane
            for rr in range(_GROUP):
                for q in range(_NE // _GROUP):
                    obuf[g * _GROUP + rr, pl.ds(q * _GROUP, _GROUP)] = zeros
            m1 = neginf
            m2 = neginf
            i1 = jnp.zeros((_GROUP,), jnp.int32)
            i2 = jnp.zeros((_GROUP,), jnp.int32)
            for j in range(_NE):
                jv = (lane + j) & (_NE - 1)
                v = plsc.load_gather(buf, [rows, jv])
                b1 = (v > m1) | ((v == m1) & (jv < i1))
                b2 = (v > m2) | ((v == m2) & (jv < i2))
                m2 = jnp.where(b1, m1, jnp.where(b2, v, m2))
                i2 = jnp.where(b1, i1, jnp.where(b2, jv, i2))
                m1 = jnp.where(b1, v, m1)
                i1 = jnp.where(b1, jv, i1)
            r = jnp.exp(m2 - m1)
            recip = 1.0 / (1.0 + r)
            plsc.store_scatter(obuf, [rows, i1], recip)
            plsc.store_scatter(obuf, [rows, i2], r * recip)
            return inner

        lax.fori_loop(0, _CHUNK // _GROUP, group_step, 0)
        pltpu.sync_copy(obuf, out_hbm.at[pl.ds(base, _CHUNK)])
        return carry

    lax.fori_loop(0, _ROWS_PER_W // _CHUNK, chunk_step, 0)


def _route(s):
    mesh = plsc.VectorSubcoreMesh(core_axis_name="c", subcore_axis_name="s")
    f = functools.partial(
        pl.kernel,
        mesh=mesh,
        out_type=jax.ShapeDtypeStruct((_T, _NE), jnp.float32),
        scratch_types=[
            pltpu.VMEM((_CHUNK, _NE), jnp.float32),
            pltpu.VMEM((_CHUNK, _NE), jnp.float32),
        ],
        compiler_params=pltpu.CompilerParams(needs_layout_passes=False),
    )(_route_body)
    return f(s)


def kernel(x, W, b):
    return _route(_logits(x, W, b))


# speedup vs baseline: 1.8412x; 1.3575x over previous
import jax
import jax.numpy as jnp
from jax import lax
from jax.experimental import pallas as pl

_D = 768
_NE = 64
_BT = 4096


def _gate_rows(s):
    col = lax.broadcasted_iota(jnp.int32, s.shape, 1)
    colf = col.astype(jnp.float32)
    ui = lax.bitcast_convert_type(s, jnp.int32)
    idxbits = jnp.where(s < 0.0, col, _NE - 1 - col)
    kf = lax.bitcast_convert_type((ui & -_NE) | idxbits, jnp.float32)
    k1 = jnp.max(kf, axis=-1, keepdims=True)
    is1 = kf == k1
    s2 = jnp.where(is1, -jnp.inf, s)
    m2 = jnp.max(s2, axis=-1, keepdims=True)
    i2 = jnp.min(jnp.where(s2 == m2, colf, float(_NE)), axis=-1, keepdims=True)
    mask = is1 | (colf == i2)
    e = jnp.exp(s - k1)
    c = jnp.where(mask, 1.0 + 1e-8, 1e-8)
    denom = jnp.sum(e * c, axis=-1, keepdims=True)
    return jnp.where(mask, e, 0.0) * (1.0 / denom)


def _fused_body(x_ref, w_ref, b_ref, o_ref, acc_ref):
    from jax.experimental.pallas import tpu as pltpu

    k = pl.program_id(1)
    part = lax.dot_general(
        x_ref[...], w_ref[...],
        (((1,), (1,)), ((), ())),
        preferred_element_type=jnp.float32,
    )

    @pl.when(k == 0)
    def _():
        acc_ref[...] = part

    @pl.when(k == 1)
    def _():
        o_ref[...] = _gate_rows(acc_ref[...] + part + b_ref[...])


def kernel(x, W, b):
    from jax.experimental.pallas import tpu as pltpu

    t = x.shape[0]
    b2 = b.reshape(1, _NE)
    hd = _D // 2
    return pl.pallas_call(
        _fused_body,
        grid=(t // _BT, 2),
        in_specs=[
            pl.BlockSpec((_BT, hd), lambda i, k: (i, k)),
            pl.BlockSpec((_NE, hd), lambda i, k: (0, k)),
            pl.BlockSpec((1, _NE), lambda i, k: (0, 0)),
        ],
        out_specs=pl.BlockSpec((_BT, _NE), lambda i, k: (i, 0)),
        out_shape=jax.ShapeDtypeStruct((t, _NE), jnp.float32),
        scratch_shapes=[pltpu.VMEM((_BT, _NE), jnp.float32)],
    )(x, W, b2)


# speedup vs baseline: 1.9960x; 1.0841x over previous
import jax
import jax.numpy as jnp
from jax import lax
from jax.experimental import pallas as pl

_D = 768
_NE = 64
_BT = 4096


def _gate_rows(s):
    col = lax.broadcasted_iota(jnp.int32, s.shape, 1)
    colf = col.astype(jnp.float32)
    ui = lax.bitcast_convert_type(s, jnp.int32)
    idxbits = jnp.where(s < 0.0, col, _NE - 1 - col)
    kf = lax.bitcast_convert_type((ui & -_NE) | idxbits, jnp.float32)
    k1 = jnp.max(kf, axis=-1, keepdims=True)
    is1 = kf == k1
    s2 = jnp.where(is1, -jnp.inf, s)
    m2 = jnp.max(s2, axis=-1, keepdims=True)
    i2 = jnp.min(jnp.where(s2 == m2, colf, float(_NE)), axis=-1, keepdims=True)
    mask = is1 | (colf == i2)
    e = jnp.exp(s - k1)
    c = jnp.where(mask, 1.0 + 1e-8, 1e-8)
    denom = jnp.sum(e * c, axis=-1, keepdims=True)
    return jnp.where(mask, e, 0.0) * (1.0 / denom)


def _fused_body(x1_ref, x2_ref, w_ref, b_ref, o_ref):
    w = w_ref[...]
    bb = b_ref[...]
    dn = (((1,), (1,)), ((), ()))
    s1 = lax.dot_general(x1_ref[0], w, dn, preferred_element_type=jnp.float32)
    o_ref[0] = _gate_rows(s1 + bb)
    s2 = lax.dot_general(x2_ref[0], w, dn, preferred_element_type=jnp.float32)
    o_ref[1] = _gate_rows(s2 + bb)


def kernel(x, W, b):
    t = x.shape[0]
    half = t // 2
    bt = _BT // 2
    x3 = x.reshape(2, half, _D)
    b2 = b.reshape(1, _NE)
    out = pl.pallas_call(
        _fused_body,
        grid=(half // bt,),
        in_specs=[
            pl.BlockSpec((1, bt, _D), lambda i: (0, i, 0)),
            pl.BlockSpec((1, bt, _D), lambda i: (1, i, 0)),
            pl.BlockSpec((_NE, _D), lambda i: (0, 0)),
            pl.BlockSpec((1, _NE), lambda i: (0, 0)),
        ],
        out_specs=pl.BlockSpec((2, bt, _NE), lambda i: (0, i, 0)),
        out_shape=jax.ShapeDtypeStruct((2, half, _NE), jnp.float32),
    )(x3, x3, W, b2)
    return out.reshape(t, _NE)


# speedup vs baseline: 2.5161x; 1.2605x over previous
import jax
import jax.numpy as jnp
from jax import lax
from jax.experimental import pallas as pl

_D = 768
_NE = 64
_BT = 4096


def _gate_rows(s):
    col = lax.broadcasted_iota(jnp.int32, s.shape, 1)
    colf = col.astype(jnp.float32)
    ui = lax.bitcast_convert_type(s, jnp.int32)
    idxbits = jnp.where(s < 0.0, col, _NE - 1 - col)
    kf = lax.bitcast_convert_type((ui & -_NE) | idxbits, jnp.float32)
    k1 = jnp.max(kf, axis=-1, keepdims=True)
    is1 = kf == k1
    s2 = jnp.where(is1, -jnp.inf, s)
    m2 = jnp.max(s2, axis=-1, keepdims=True)
    i2 = jnp.min(jnp.where(s2 == m2, colf, float(_NE)), axis=-1, keepdims=True)
    mask = is1 | (colf == i2)
    e = jnp.exp(s - k1)
    c = jnp.where(mask, 1.0 + 1e-8, 1e-8)
    denom = jnp.sum(e * c, axis=-1, keepdims=True)
    return jnp.where(mask, e, 0.0) * (1.0 / denom)


def _fused_body(x_ref, w_ref, b_ref, o_ref):
    s = lax.dot_general(
        x_ref[...], w_ref[...],
        (((1,), (1,)), ((), ())),
        preferred_element_type=jnp.float32,
    )
    s = s + b_ref[...]
    o_ref[...] = _gate_rows(s)


def kernel(x, W, b):
    t = x.shape[0]
    b2 = b.reshape(1, _NE)
    return pl.pallas_call(
        _fused_body,
        grid=(t // _BT,),
        in_specs=[
            pl.BlockSpec((_BT, _D), lambda i: (i, 0)),
            pl.BlockSpec((_NE, _D), lambda i: (0, 0)),
            pl.BlockSpec((1, _NE), lambda i: (0, 0)),
        ],
        out_specs=pl.BlockSpec((_BT, _NE), lambda i: (i, 0)),
        out_shape=jax.ShapeDtypeStruct((t, _NE), jnp.float32),
    )(x, W, b2)
